# Initial kernel scaffold; baseline (speedup 1.0000x reference)
import jax
import jax.numpy as jnp
from jax.experimental import pallas as pl


def kernel(class_idx, image, emb_table):
    raise NotImplementedError("write your pallas kernel here")


# speedup vs baseline: 1.3001x; 1.3001x over previous
import jax
import jax.numpy as jnp
from jax.experimental import pallas as pl
from jax.experimental.pallas import tpu as pltpu

_B, _C, _H, _W = 8, 96, 224, 224
_E = 64
_HB = 16


def _body(idx_ref, emb_row_ref, img_ref, out_ref):
    row = emb_row_ref[0, 0, :]
    out_ref[0, :_E] = jnp.broadcast_to(row[:, None, None], (_E, _HB, _W))
    out_ref[0, _E:] = img_ref[0]


def kernel(class_idx, image, emb_table):
    grid = (_B, _H // _HB)
    return pl.pallas_call(
        _body,
        grid_spec=pltpu.PrefetchScalarGridSpec(
            num_scalar_prefetch=1,
            grid=grid,
            in_specs=[
                pl.BlockSpec((1, 1, _E), lambda b, h, idx_ref: (idx_ref[b], 0, 0)),
                pl.BlockSpec((1, _C, _HB, _W), lambda b, h, idx_ref: (b, 0, h, 0)),
            ],
            out_specs=pl.BlockSpec((1, _C + _E, _HB, _W),
                                   lambda b, h, idx_ref: (b, 0, h, 0)),
        ),
        out_shape=jax.ShapeDtypeStruct((_B, _C + _E, _H, _W), jnp.float32),
    )(class_idx, emb_table.reshape(-1, 1, _E), image)


# speedup vs baseline: 1.4367x; 1.1051x over previous
import jax
import jax.numpy as jnp
from jax.experimental import pallas as pl
from jax.experimental.pallas import tpu as pltpu

_B, _C, _H, _W = 8, 96, 224, 224
_E = 64
_HB = 32


def _body(idx_ref, emb_row_ref, img_ref, out_ref):
    row = emb_row_ref[0, 0, :]
    out_ref[0, :_E] = jnp.broadcast_to(row[:, None, None], (_E, _HB, _W))
    out_ref[0, _E:] = img_ref[0]


def kernel(class_idx, image, emb_table):
    grid = (_B, _H // _HB)
    return pl.pallas_call(
        _body,
        grid_spec=pltpu.PrefetchScalarGridSpec(
            num_scalar_prefetch=1,
            grid=grid,
            in_specs=[
                pl.BlockSpec((1, 1, _E), lambda b, h, idx_ref: (idx_ref[b], 0, 0)),
                pl.BlockSpec((1, _C, _HB, _W), lambda b, h, idx_ref: (b, 0, h, 0)),
            ],
            out_specs=pl.BlockSpec((1, _C + _E, _HB, _W),
                                   lambda b, h, idx_ref: (b, 0, h, 0)),
        ),
        out_shape=jax.ShapeDtypeStruct((_B, _C + _E, _H, _W), jnp.float32),
    )(class_idx, emb_table.reshape(-1, 1, _E), image)


# speedup vs baseline: 1.4697x; 1.0230x over previous
import jax
import jax.numpy as jnp
from jax.experimental import pallas as pl
from jax.experimental.pallas import tpu as pltpu

_B, _C, _H, _W = 8, 96, 224, 224
_E = 64
_HB = 56


def _body(idx_ref, emb_row_ref, img_ref, out_ref):
    row = emb_row_ref[0, 0, :]
    out_ref[0, :_E] = jnp.broadcast_to(row[:, None, None], (_E, _HB, _W))
    out_ref[0, _E:] = img_ref[0]


def kernel(class_idx, image, emb_table):
    grid = (_B, _H // _HB)
    return pl.pallas_call(
        _body,
        grid_spec=pltpu.PrefetchScalarGridSpec(
            num_scalar_prefetch=1,
            grid=grid,
            in_specs=[
                pl.BlockSpec((1, 1, _E), lambda b, h, idx_ref: (idx_ref[b], 0, 0)),
                pl.BlockSpec((1, _C, _HB, _W), lambda b, h, idx_ref: (b, 0, h, 0)),
            ],
            out_specs=pl.BlockSpec((1, _C + _E, _HB, _W),
                                   lambda b, h, idx_ref: (b, 0, h, 0)),
        ),
        out_shape=jax.ShapeDtypeStruct((_B, _C + _E, _H, _W), jnp.float32),
    )(class_idx, emb_table.reshape(-1, 1, _E), image)


# speedup vs baseline: 1.5050x; 1.0240x over previous
import jax
import jax.numpy as jnp
from jax.experimental import pallas as pl
from jax.experimental.pallas import tpu as pltpu

_B, _C, _H, _W = 8, 96, 224, 224
_E = 64
_HB = 112


def _body(idx_ref, emb_row_ref, img_ref, out_ref):
    row = emb_row_ref[0, 0, :]
    out_ref[0, :_E] = jnp.broadcast_to(row[:, None, None], (_E, _HB, _W))
    out_ref[0, _E:] = img_ref[0]


def kernel(class_idx, image, emb_table):
    grid = (_B, _H // _HB)
    return pl.pallas_call(
        _body,
        grid_spec=pltpu.PrefetchScalarGridSpec(
            num_scalar_prefetch=1,
            grid=grid,
            in_specs=[
                pl.BlockSpec((1, 1, _E), lambda b, h, idx_ref: (idx_ref[b], 0, 0)),
                pl.BlockSpec((1, _C, _HB, _W), lambda b, h, idx_ref: (b, 0, h, 0)),
            ],
            out_specs=pl.BlockSpec((1, _C + _E, _HB, _W),
                                   lambda b, h, idx_ref: (b, 0, h, 0)),
        ),
        out_shape=jax.ShapeDtypeStruct((_B, _C + _E, _H, _W), jnp.float32),
    )(class_idx, emb_table.reshape(-1, 1, _E), image)
